# baseline (speedup 1.0000x reference)
import jax
import jax.numpy as jnp
from jax.experimental import pallas as pl


def _pass1_kernel(a_ref, x_ref, xb_ref, y_ref):
    y_ref[...] = (
        jnp.dot(a_ref[...], x_ref[...], preferred_element_type=jnp.float32)
        + xb_ref[...]
    )


def _pass2_kernel(a_ref, y_ref, o_ref):
    o_ref[...] = jnp.dot(a_ref[...], y_ref[...], preferred_element_type=jnp.float32)


def _pick_block(n):
    for bm in (400, 200, 80, 40, 16, 8):
        if n % bm == 0:
            return bm
    return n


def kernel(x, A):
    n, d = x.shape
    bm = _pick_block(n)
    nm = n // bm

    y = pl.pallas_call(
        _pass1_kernel,
        grid=(nm,),
        in_specs=[
            pl.BlockSpec((bm, n), lambda m: (m, 0)),
            pl.BlockSpec((n, d), lambda m: (0, 0)),
            pl.BlockSpec((bm, d), lambda m: (m, 0)),
        ],
        out_specs=pl.BlockSpec((bm, d), lambda m: (m, 0)),
        out_shape=jax.ShapeDtypeStruct((n, d), jnp.float32),
    )(A, x, x)

    out = pl.pallas_call(
        _pass2_kernel,
        grid=(nm,),
        in_specs=[
            pl.BlockSpec((bm, n), lambda m: (m, 0)),
            pl.BlockSpec((n, d), lambda m: (0, 0)),
        ],
        out_specs=pl.BlockSpec((bm, d), lambda m: (m, 0)),
        out_shape=jax.ShapeDtypeStruct((n, d), jnp.float32),
    )(A, y)
    return out
